# Initial kernel scaffold; baseline (speedup 1.0000x reference)
import jax
import jax.numpy as jnp
from jax.experimental import pallas as pl


def kernel(vocab_ids, table):
    raise NotImplementedError("write your pallas kernel here")


# speedup vs baseline: 3.3228x; 3.3228x over previous
import functools

import jax
import jax.numpy as jnp
from jax import lax
from jax.experimental import pallas as pl
from jax.experimental.pallas import tpu as pltpu
from jax.experimental.pallas import tpu_sc as plsc

_CHUNK = 128
_NBUF = 5


def kernel(vocab_ids, table):
    bsz, hist = vocab_ids.shape
    _, d = table.shape
    n = bsz * hist

    info = plsc.get_sparse_core_info()
    nw = info.num_cores * info.num_subcores
    n_per_w = n // nw
    n_chunks = n_per_w // _CHUNK
    n_groups = n_chunks // _NBUF
    assert n_per_w * nw == n and n_chunks * _CHUNK == n_per_w
    assert n_groups * _NBUF == n_chunks

    idx = vocab_ids.astype(jnp.int32).reshape(nw, n_chunks, _CHUNK)

    mesh = plsc.VectorSubcoreMesh(core_axis_name="c", subcore_axis_name="s")

    @functools.partial(
        pl.kernel,
        out_type=jax.ShapeDtypeStruct((n, d), table.dtype),
        mesh=mesh,
        scratch_types=[
            pltpu.VMEM((n_chunks, _CHUNK), jnp.int32),
            pltpu.VMEM((_NBUF, _CHUNK, d), jnp.float32),
            pltpu.SemaphoreType.DMA((_NBUF,)),
            pltpu.SemaphoreType.DMA((_NBUF,)),
        ],
    )
    def emb_lookup(idx_hbm, table_hbm, out_hbm, idx_v, bufs, gsem, wsem):
        wid = lax.axis_index("s") * info.num_cores + lax.axis_index("c")
        row0 = wid * n_per_w
        pltpu.sync_copy(idx_hbm.at[wid], idx_v)

        def gather(chunk, b):
            return pltpu.make_async_copy(
                table_hbm.at[idx_v.at[chunk]], bufs.at[b], gsem.at[b])

        def write(chunk, b):
            return pltpu.make_async_copy(
                bufs.at[b], out_hbm.at[pl.ds(row0 + chunk * _CHUNK, _CHUNK)],
                wsem.at[b])

        for b in range(_NBUF):
            gather(b, b).start()

        @pl.loop(0, n_groups)
        def _(g):
            c0 = g * _NBUF
            for b in range(_NBUF):
                gather(c0 + b, b).wait()
                write(c0 + b, b).start()
            for b in range(_NBUF):
                write(c0 + b, b).wait()
                nxt = c0 + _NBUF + b

                @pl.when(nxt < n_chunks)
                def _():
                    gather(nxt, b).start()

    out = emb_lookup(idx, table)
    return out.reshape(bsz, hist, d)
